# Initial kernel scaffold; baseline (speedup 1.0000x reference)
import jax
import jax.numpy as jnp
from jax.experimental import pallas as pl


def kernel(x, enc_w1, enc_b1, enc_w2, enc_b2, enc_w3, enc_b3, codebook, dec_w1, dec_b1, dec_w2, dec_b2, dec_w3, dec_b3):
    raise NotImplementedError("write your pallas kernel here")


# speedup vs baseline: 1.0629x; 1.0629x over previous
import functools

import jax
import jax.numpy as jnp
from jax.experimental import pallas as pl


def _conv(x, w, b, stride, pad):
    y = jax.lax.conv_general_dilated(x, w, window_strides=(stride, stride),
                                     padding=((pad, pad), (pad, pad)),
                                     dimension_numbers=('NCHW', 'OIHW', 'NCHW'))
    return y + b[None, :, None, None]


def _convT(x, w, b, stride, pad):
    y = jax.lax.conv_transpose(x, w, strides=(stride, stride),
                               padding=((pad, pad), (pad, pad)),
                               dimension_numbers=('NCHW', 'OIHW', 'NCHW'))
    return y + b[None, :, None, None]


_ROWS = 512


def _vq_kernel(z_ref, cb_ref, quant_ref, idx_ref, loss_ref):
    z = z_ref[...]
    cb = cb_ref[...]
    zz = jnp.sum(z * z, axis=1, keepdims=True)
    cc = jnp.sum(cb * cb, axis=1)[None, :]
    cross = jax.lax.dot_general(
        z, cb, (((1,), (1,)), ((), ())),
        preferred_element_type=jnp.float32)
    d2 = zz + cc - 2.0 * cross
    idx = jnp.argmin(d2, axis=1).astype(jnp.int32)
    k = d2.shape[1]
    onehot = (idx[:, None] == jax.lax.broadcasted_iota(jnp.int32, (1, k), 1)
              ).astype(jnp.float32)
    quant = jax.lax.dot_general(
        onehot, cb, (((1,), (0,)), ((), ())),
        preferred_element_type=jnp.float32)
    quant_ref[...] = quant
    idx_ref[...] = idx.reshape(1, 1, -1)
    diff = quant - z
    part = jnp.sum(diff * diff).reshape(1, 1)
    @pl.when(pl.program_id(0) == 0)
    def _():
        loss_ref[...] = jnp.zeros((1, 1), jnp.float32)
    loss_ref[...] += part


def _vq(z_flat, codebook):
    n, d = z_flat.shape
    k = codebook.shape[0]
    nblk = n // _ROWS
    quant, idx, losssum = pl.pallas_call(
        _vq_kernel,
        grid=(nblk,),
        in_specs=[
            pl.BlockSpec((_ROWS, d), lambda i: (i, 0)),
            pl.BlockSpec((k, d), lambda i: (0, 0)),
        ],
        out_specs=[
            pl.BlockSpec((_ROWS, d), lambda i: (i, 0)),
            pl.BlockSpec((1, 1, _ROWS), lambda i: (i, 0, 0)),
            pl.BlockSpec((1, 1), lambda i: (0, 0)),
        ],
        out_shape=[
            jax.ShapeDtypeStruct((n, d), jnp.float32),
            jax.ShapeDtypeStruct((nblk, 1, _ROWS), jnp.int32),
            jax.ShapeDtypeStruct((1, 1), jnp.float32),
        ],
    )(z_flat, codebook)
    return quant, idx.reshape(n), losssum[0, 0]


def kernel(x, enc_w1, enc_b1, enc_w2, enc_b2, enc_w3, enc_b3, codebook,
           dec_w1, dec_b1, dec_w2, dec_b2, dec_w3, dec_b3):
    beta = 0.25
    h = jax.nn.relu(_conv(x, enc_w1, enc_b1, 2, 1))
    h = jax.nn.relu(_conv(h, enc_w2, enc_b2, 2, 1))
    z = _conv(h, enc_w3, enc_b3, 1, 1)
    B, D, Hh, Ww = z.shape
    z_flat = jnp.transpose(z, (0, 2, 3, 1)).reshape(-1, D)
    quant_flat, indices, losssum = _vq(z_flat, codebook)
    quantized = jnp.transpose(quant_flat.reshape(B, Hh, Ww, D), (0, 3, 1, 2))
    codebook_loss = losssum / jnp.float32(z_flat.size)
    commitment_loss = beta * codebook_loss
    g = jax.nn.relu(_conv(quantized, dec_w1, dec_b1, 1, 1))
    g = jax.nn.relu(_convT(g, dec_w2, dec_b2, 2, 1))
    x_recon = _convT(g, dec_w3, dec_b3, 2, 1)
    return (x_recon, codebook_loss, commitment_loss,
            indices.reshape(B, Hh, Ww))
